# Initial kernel scaffold; baseline (speedup 1.0000x reference)
import jax
import jax.numpy as jnp
from jax.experimental import pallas as pl


def kernel(mem_pred, mem_gt):
    raise NotImplementedError("write your pallas kernel here")


# speedup vs baseline: 17.6547x; 17.6547x over previous
import jax
import jax.numpy as jnp
from jax.experimental import pallas as pl
from jax.experimental.pallas import tpu as pltpu

_N = 4096
_BK = 256
_NBLK = _N // _BK
_LBD = 1.0


def _body(pred_row, gt_row, pred_col, gt_col, out_ref, acc_ref):
    i = pl.program_id(0)
    n = jnp.float32(_N)

    @pl.when(i == 0)
    def _init():
        x = pred_row[:, :]
        g = gt_row[:, :]
        sum_x = jnp.sum(x)
        sum_x2 = jnp.sum(x * x)
        pos = jax.lax.broadcasted_iota(jnp.int32, (1, _N), 1).astype(jnp.float32)
        s1 = jnp.sum(x * (2.0 * pos - (n - 1.0)))
        ss = n * sum_x2 - sum_x * sum_x
        var = (ss - s1 * s1 / (n * n)) / (n * n - 1.0)
        scale = 6.8 / jnp.sqrt(var)
        l1 = jnp.sum(jnp.abs(x - g)) / n
        acc_ref[0] = scale
        acc_ref[1] = l1
        acc_ref[2] = 0.0

    scale = acc_ref[0]
    xj = pred_row[:, :]
    xk = pred_col[:, :]
    z = (xj - xk) * scale
    p = 1.0 / (1.0 + jnp.exp(-z))
    sig_sum = jnp.sum(p, axis=1, keepdims=True)
    r = (sig_sum + 0.5) / n

    gj = gt_row[:, :]
    gk = gt_col[:, :]
    cnt = jnp.where(gj < gk, 1.0, 0.0) + jnp.where(gj <= gk, 1.0, 0.0)
    lr = jnp.sum(cnt, axis=1, keepdims=True)
    grank = (n + 1.0 - (lr + 1.0) * 0.5) / n

    d = r - grank
    acc_ref[2] += jnp.sum(d * d)

    @pl.when(i == _NBLK - 1)
    def _fin():
        out_ref[0] = acc_ref[2] / n + _LBD * acc_ref[1]


def kernel(mem_pred, mem_gt):
    pred_row = mem_pred.reshape(1, _N)
    gt_row = mem_gt.reshape(1, _N)
    pred_col = mem_pred.reshape(_N, 1)
    gt_col = mem_gt.reshape(_N, 1)

    out = pl.pallas_call(
        _body,
        grid=(_NBLK,),
        in_specs=[
            pl.BlockSpec((1, _N), lambda i: (0, 0)),
            pl.BlockSpec((1, _N), lambda i: (0, 0)),
            pl.BlockSpec((_BK, 1), lambda i: (i, 0)),
            pl.BlockSpec((_BK, 1), lambda i: (i, 0)),
        ],
        out_specs=pl.BlockSpec(memory_space=pltpu.SMEM),
        out_shape=jax.ShapeDtypeStruct((1,), jnp.float32),
        scratch_shapes=[pltpu.SMEM((3,), jnp.float32)],
    )(pred_row, gt_row, pred_col, gt_col)
    return out[0]


# speedup vs baseline: 21.6250x; 1.2249x over previous
import jax
import jax.numpy as jnp
from jax.experimental import pallas as pl
from jax.experimental.pallas import tpu as pltpu

_N = 4096
_BK = 256
_NBLK = _N // _BK
_LBD = 1.0


def _body(pred_row, gt_row, pred_col, gt_col, out_ref, xs_row, acc_ref):
    i = pl.program_id(0)
    n = jnp.float32(_N)

    @pl.when(i == 0)
    def _init():
        x = pred_row[:, :]
        g = gt_row[:, :]
        sum_x = jnp.sum(x)
        sum_x2 = jnp.sum(x * x)
        pos = jax.lax.broadcasted_iota(jnp.int32, (1, _N), 1).astype(jnp.float32)
        s1 = jnp.sum(x * (2.0 * pos - (n - 1.0)))
        ss = n * sum_x2 - sum_x * sum_x
        var = (ss - s1 * s1 / (n * n)) / (n * n - 1.0)
        s2 = 3.4 / jnp.sqrt(var)
        acc_ref[0] = s2
        acc_ref[1] = jnp.sum(jnp.abs(x - g)) / n
        acc_ref[2] = 0.0
        xs_row[:, :] = x * s2

    s2 = acc_ref[0]
    xsj = xs_row[:, :]
    xsk = pred_col[:, :] * s2
    gj = gt_row[:, :]
    gk = gt_col[:, :]
    w = jnp.tanh(xsj - xsk) - jnp.sign(gj - gk)
    wsum = jnp.sum(w, axis=1, keepdims=True)
    acc_ref[2] += jnp.sum(wsum * wsum)

    @pl.when(i == _NBLK - 1)
    def _fin():
        out_ref[0] = acc_ref[2] / (4.0 * n * n * n) + _LBD * acc_ref[1]


def kernel(mem_pred, mem_gt):
    pred_row = mem_pred.reshape(1, _N)
    gt_row = mem_gt.reshape(1, _N)
    pred_col = mem_pred.reshape(_N, 1)
    gt_col = mem_gt.reshape(_N, 1)

    out = pl.pallas_call(
        _body,
        grid=(_NBLK,),
        in_specs=[
            pl.BlockSpec((1, _N), lambda i: (0, 0)),
            pl.BlockSpec((1, _N), lambda i: (0, 0)),
            pl.BlockSpec((_BK, 1), lambda i: (i, 0)),
            pl.BlockSpec((_BK, 1), lambda i: (i, 0)),
        ],
        out_specs=pl.BlockSpec(memory_space=pltpu.SMEM),
        out_shape=jax.ShapeDtypeStruct((1,), jnp.float32),
        scratch_shapes=[
            pltpu.VMEM((1, _N), jnp.float32),
            pltpu.SMEM((3,), jnp.float32),
        ],
    )(pred_row, gt_row, pred_col, gt_col)
    return out[0]


# speedup vs baseline: 23.6148x; 1.0920x over previous
import jax
import jax.numpy as jnp
from jax.experimental import pallas as pl
from jax.experimental.pallas import tpu as pltpu

_N = 4096
_BK = 256
_NBLK = _N // _BK
_LBD = 1.0


def _body(pred_row, gt_row, pred_col, gt_col, out_ref, xs_row, acc_ref):
    i = pl.program_id(0)
    n = jnp.float32(_N)

    @pl.when(i == 0)
    def _init():
        x = pred_row[:, :]
        g = gt_row[:, :]
        sum_x = jnp.sum(x)
        sum_x2 = jnp.sum(x * x)
        pos = jax.lax.broadcasted_iota(jnp.int32, (1, _N), 1).astype(jnp.float32)
        s1 = jnp.sum(x * (2.0 * pos - (n - 1.0)))
        ss = n * sum_x2 - sum_x * sum_x
        var = (ss - s1 * s1 / (n * n)) / (n * n - 1.0)
        s2 = 3.4 / jnp.sqrt(var)
        acc_ref[0] = s2
        acc_ref[1] = jnp.sum(jnp.abs(x - g)) / n
        acc_ref[2] = 0.0
        xs_row[:, :] = x * s2

    s2 = acc_ref[0]
    xsj = xs_row[:, :]
    xsk = pred_col[:, :] * s2
    gj = gt_row[:, :]
    gk = gt_col[:, :]
    d = gj - gk
    sgn = jnp.minimum(jnp.maximum(d * 1e25, -1.0), 1.0)
    sgn = jnp.minimum(jnp.maximum(sgn * 1e25, -1.0), 1.0)
    w = jnp.tanh(xsj - xsk) - sgn
    wsum = jnp.sum(w, axis=1, keepdims=True)
    acc_ref[2] += jnp.sum(wsum * wsum)

    @pl.when(i == _NBLK - 1)
    def _fin():
        out_ref[0] = acc_ref[2] / (4.0 * n * n * n) + _LBD * acc_ref[1]


def kernel(mem_pred, mem_gt):
    pred_row = mem_pred.reshape(1, _N)
    gt_row = mem_gt.reshape(1, _N)
    pred_col = mem_pred.reshape(_N, 1)
    gt_col = mem_gt.reshape(_N, 1)

    out = pl.pallas_call(
        _body,
        grid=(_NBLK,),
        in_specs=[
            pl.BlockSpec((1, _N), lambda i: (0, 0)),
            pl.BlockSpec((1, _N), lambda i: (0, 0)),
            pl.BlockSpec((_BK, 1), lambda i: (i, 0)),
            pl.BlockSpec((_BK, 1), lambda i: (i, 0)),
        ],
        out_specs=pl.BlockSpec(memory_space=pltpu.SMEM),
        out_shape=jax.ShapeDtypeStruct((1,), jnp.float32),
        scratch_shapes=[
            pltpu.VMEM((1, _N), jnp.float32),
            pltpu.SMEM((3,), jnp.float32),
        ],
    )(pred_row, gt_row, pred_col, gt_col)
    return out[0]
